# Initial kernel scaffold; baseline (speedup 1.0000x reference)
import jax
import jax.numpy as jnp
from jax.experimental import pallas as pl


def kernel(fmap, wh, reg, K):
    raise NotImplementedError("write your pallas kernel here")


# speedup vs baseline: 10.1056x; 10.1056x over previous
import functools

import jax
import jax.numpy as jnp
from jax import lax
from jax.experimental import pallas as pl
from jax.experimental.pallas import tpu as pltpu

B = 8
C = 80
H = 128
W = 128
HW = H * W
K_STATIC = 100
KPAD = 128
CB = 4
CBLK = C // CB


def _nms_topk_body(fmap_ref, scores_ref, inds_ref, masked_ref, rowmax_ref):
    cb = pl.program_id(1)
    x = fmap_ref[0]
    neg = jnp.float32(-jnp.inf)
    left = jnp.concatenate([x[:, :, 1:], jnp.full((CBLK, H, 1), neg)], axis=2)
    right = jnp.concatenate([jnp.full((CBLK, H, 1), neg), x[:, :, :-1]], axis=2)
    mw = jnp.maximum(jnp.maximum(left, right), x)
    up = jnp.concatenate([mw[:, 1:, :], jnp.full((CBLK, 1, W), neg)], axis=1)
    down = jnp.concatenate([jnp.full((CBLK, 1, W), neg), mw[:, :-1, :]], axis=1)
    m9 = jnp.maximum(jnp.maximum(up, down), mw)
    masked = jnp.where(m9 == x, x, jnp.float32(0.0))
    masked_ref[pl.ds(cb * CBLK * H, CBLK * H), :] = masked.reshape(CBLK * H, W)
    rowmax_ref[pl.ds(cb * CBLK, CBLK), :] = masked.max(axis=2)

    @pl.when(cb == CB - 1)
    def _extract():
        scores_ref[...] = jnp.zeros((1, 1, KPAD), jnp.float32)
        inds_ref[...] = jnp.zeros((1, 1, KPAD), jnp.int32)
        rpos = (lax.broadcasted_iota(jnp.int32, (C, H), 0) * H
                + lax.broadcasted_iota(jnp.int32, (C, H), 1))
        lane = lax.broadcasted_iota(jnp.int32, (1, W), 1)

        def body(i, _):
            rm = rowmax_ref[...]
            m = jnp.max(rm)
            p = jnp.min(jnp.where(rm == m, rpos, jnp.int32(C * H)))
            c_i = p // H
            h_i = p - c_i * H
            row = masked_ref[pl.ds(p, 1), :]
            col = jnp.min(jnp.where(row == m, lane, jnp.int32(W)))
            lane3 = lane.reshape(1, 1, KPAD)
            scores_ref[...] = jnp.where(lane3 == i, m, scores_ref[...])
            inds_ref[...] = jnp.where(lane3 == i, p * W + col, inds_ref[...])
            newrow = jnp.where(lane == col, jnp.float32(-1.0), row)
            masked_ref[pl.ds(p, 1), :] = newrow
            rmrow = rowmax_ref[pl.ds(c_i, 1), :]
            rowmax_ref[pl.ds(c_i, 1), :] = jnp.where(
                lane == h_i, jnp.max(newrow), rmrow)
            return 0

        lax.fori_loop(0, K_STATIC, body, 0)


def _nms_topk(fmap):
    return pl.pallas_call(
        _nms_topk_body,
        grid=(B, CB),
        in_specs=[pl.BlockSpec((1, CBLK, H, W), lambda b, cb: (b, cb, 0, 0))],
        out_specs=[
            pl.BlockSpec((1, 1, KPAD), lambda b, cb: (b, 0, 0)),
            pl.BlockSpec((1, 1, KPAD), lambda b, cb: (b, 0, 0)),
        ],
        out_shape=[
            jax.ShapeDtypeStruct((B, 1, KPAD), jnp.float32),
            jax.ShapeDtypeStruct((B, 1, KPAD), jnp.int32),
        ],
        scratch_shapes=[
            pltpu.VMEM((C * H, W), jnp.float32),
            pltpu.VMEM((C, H), jnp.float32),
        ],
    )(fmap)


def _decode_jnp(wh, reg, scores, inds):
    sp = inds % HW
    cls = (inds // HW).astype(jnp.float32)
    ys = (sp // W).astype(jnp.float32)
    xs = (sp % W).astype(jnp.float32)
    wh_p = wh.reshape(B, 2, HW)
    reg_p = reg.reshape(B, 2, HW)
    wx = jnp.take_along_axis(wh_p[:, 0, :], sp, axis=1)
    wy = jnp.take_along_axis(wh_p[:, 1, :], sp, axis=1)
    rx = jnp.take_along_axis(reg_p[:, 0, :], sp, axis=1)
    ry = jnp.take_along_axis(reg_p[:, 1, :], sp, axis=1)
    xs = xs + rx
    ys = ys + ry
    x1 = xs - wx / 2
    y1 = ys - wy / 2
    x2 = xs + wx / 2
    y2 = ys + wy / 2
    return x1, y1, x2, y2, cls


def kernel(fmap, wh, reg, K):
    scores, inds = _nms_topk(fmap)
    scores = scores.reshape(B, KPAD)
    inds = inds.reshape(B, KPAD)
    x1, y1, x2, y2, cls = _decode_jnp(wh, reg, scores, inds)
    k_zero = jnp.asarray(K, jnp.float32) - jnp.float32(K_STATIC)
    bboxes = jnp.stack([x1, y1, x2, y2], axis=2)[:, :K_STATIC, :]
    scores_out = scores[:, :K_STATIC, None] + k_zero
    clses = cls[:, :K_STATIC, None]
    return bboxes, scores_out, clses


# speedup vs baseline: 10.4529x; 1.0344x over previous
import functools

import jax
import jax.numpy as jnp
from jax import lax
from jax.experimental import pallas as pl
from jax.experimental.pallas import tpu as pltpu
from jax.experimental.pallas import tpu_sc as plsc

B = 8
C = 80
H = 128
W = 128
HW = H * W
K_STATIC = 100
KPAD = 128
CB = 4
CBLK = C // CB


def _nms_topk_body(fmap_ref, scores_ref, inds_ref, gidx_ref,
                   masked_ref, rowmax_ref):
    b = pl.program_id(0)
    cb = pl.program_id(1)
    x = fmap_ref[0]
    neg = jnp.float32(-jnp.inf)
    left = jnp.concatenate([x[:, :, 1:], jnp.full((CBLK, H, 1), neg)], axis=2)
    right = jnp.concatenate([jnp.full((CBLK, H, 1), neg), x[:, :, :-1]], axis=2)
    mw = jnp.maximum(jnp.maximum(left, right), x)
    up = jnp.concatenate([mw[:, 1:, :], jnp.full((CBLK, 1, W), neg)], axis=1)
    down = jnp.concatenate([jnp.full((CBLK, 1, W), neg), mw[:, :-1, :]], axis=1)
    m9 = jnp.maximum(jnp.maximum(up, down), mw)
    masked = jnp.where(m9 == x, x, jnp.float32(0.0))
    masked_ref[pl.ds(cb * CBLK * H, CBLK * H), :] = masked.reshape(CBLK * H, W)
    rowmax_ref[pl.ds(cb * CBLK, CBLK), :] = masked.max(axis=2)

    @pl.when(cb == CB - 1)
    def _extract():
        scores_ref[...] = jnp.zeros((1, 1, KPAD), jnp.float32)
        inds_ref[...] = jnp.zeros((1, 1, KPAD), jnp.int32)
        gidx_ref[...] = jnp.full((1, 1, KPAD), b * 2 * HW, jnp.int32)
        rpos = (lax.broadcasted_iota(jnp.int32, (C, H), 0) * H
                + lax.broadcasted_iota(jnp.int32, (C, H), 1))
        lane = lax.broadcasted_iota(jnp.int32, (1, W), 1)

        def body(i, _):
            rm = rowmax_ref[...]
            m = jnp.max(rm)
            p = jnp.min(jnp.where(rm == m, rpos, jnp.int32(C * H)))
            c_i = p // H
            h_i = p - c_i * H
            row = masked_ref[pl.ds(p, 1), :]
            col = jnp.min(jnp.where(row == m, lane, jnp.int32(W)))
            lane3 = lane.reshape(1, 1, KPAD)
            scores_ref[...] = jnp.where(lane3 == i, m, scores_ref[...])
            inds_ref[...] = jnp.where(lane3 == i, p * W + col, inds_ref[...])
            gidx_ref[...] = jnp.where(
                lane3 == i, b * 2 * HW + h_i * W + col, gidx_ref[...])
            newrow = jnp.where(lane == col, jnp.float32(-1.0), row)
            masked_ref[pl.ds(p, 1), :] = newrow
            rmrow = rowmax_ref[pl.ds(c_i, 1), :]
            rowmax_ref[pl.ds(c_i, 1), :] = jnp.where(
                lane == h_i, jnp.max(newrow), rmrow)
            return 0

        lax.fori_loop(0, K_STATIC, body, 0)


def _nms_topk(fmap):
    return pl.pallas_call(
        _nms_topk_body,
        grid=(B, CB),
        in_specs=[pl.BlockSpec((1, CBLK, H, W), lambda b, cb: (b, cb, 0, 0))],
        out_specs=[
            pl.BlockSpec((1, 1, KPAD), lambda b, cb: (b, 0, 0)),
            pl.BlockSpec((1, 1, KPAD), lambda b, cb: (b, 0, 0)),
            pl.BlockSpec((1, 1, KPAD), lambda b, cb: (b, 0, 0)),
        ],
        out_shape=[
            jax.ShapeDtypeStruct((B, 1, KPAD), jnp.float32),
            jax.ShapeDtypeStruct((B, 1, KPAD), jnp.int32),
            jax.ShapeDtypeStruct((B, 1, KPAD), jnp.int32),
        ],
        scratch_shapes=[
            pltpu.VMEM((C * H, W), jnp.float32),
            pltpu.VMEM((C, H), jnp.float32),
        ],
    )(fmap)


def _sc_decode_body(whf, regf, indsf, gidxf, x1o, y1o, x2o, y2o, clso,
                    inds_v, idxa_v, idxb_v, whx_v, why_v, rgx_v, rgy_v,
                    x1_v, y1_v, x2_v, y2_v, cls_v,
                    sem0, sem1, sem2, sem3):
    wid = lax.axis_index("s") * 2 + lax.axis_index("c")

    @pl.when(wid < B)
    def _():
        b = wid
        pltpu.sync_copy(indsf.at[pl.ds(b * KPAD, KPAD)], inds_v)
        pltpu.sync_copy(gidxf.at[pl.ds(b * KPAD, KPAD)], idxa_v)
        for j in range(KPAD // 16):
            sl = pl.ds(j * 16, 16)
            idxb_v[sl] = idxa_v[sl] + HW
        c0 = pltpu.async_copy(whf.at[idxa_v], whx_v, sem0)
        c1 = pltpu.async_copy(whf.at[idxb_v], why_v, sem1)
        c2 = pltpu.async_copy(regf.at[idxa_v], rgx_v, sem2)
        c3 = pltpu.async_copy(regf.at[idxb_v], rgy_v, sem3)
        c0.wait()
        c1.wait()
        c2.wait()
        c3.wait()
        for j in range(KPAD // 16):
            sl = pl.ds(j * 16, 16)
            ind = inds_v[sl]
            sp = lax.rem(ind, jnp.int32(HW))
            cls_v[sl] = lax.convert_element_type(
                lax.div(ind, jnp.int32(HW)), jnp.float32)
            ys = lax.convert_element_type(
                lax.div(sp, jnp.int32(W)), jnp.float32) + rgy_v[sl]
            xs = lax.convert_element_type(
                lax.rem(sp, jnp.int32(W)), jnp.float32) + rgx_v[sl]
            hw2 = whx_v[sl] * jnp.float32(0.5)
            hh2 = why_v[sl] * jnp.float32(0.5)
            x1_v[sl] = xs - hw2
            y1_v[sl] = ys - hh2
            x2_v[sl] = xs + hw2
            y2_v[sl] = ys + hh2
        pltpu.sync_copy(x1_v, x1o.at[pl.ds(b * KPAD, KPAD)])
        pltpu.sync_copy(y1_v, y1o.at[pl.ds(b * KPAD, KPAD)])
        pltpu.sync_copy(x2_v, x2o.at[pl.ds(b * KPAD, KPAD)])
        pltpu.sync_copy(y2_v, y2o.at[pl.ds(b * KPAD, KPAD)])
        pltpu.sync_copy(cls_v, clso.at[pl.ds(b * KPAD, KPAD)])


def _sc_decode(wh_flat, reg_flat, inds_flat, gidx_flat):
    f32 = jnp.float32
    fn = pl.kernel(
        _sc_decode_body,
        mesh=plsc.VectorSubcoreMesh(core_axis_name="c", subcore_axis_name="s"),
        out_type=[jax.ShapeDtypeStruct((B * KPAD,), f32)] * 5,
        scratch_types=(
            [pltpu.VMEM((KPAD,), jnp.int32)] * 3
            + [pltpu.VMEM((KPAD,), f32)] * 9
            + [pltpu.SemaphoreType.DMA] * 4
        ),
    )
    return fn(wh_flat, reg_flat, inds_flat, gidx_flat)


def kernel(fmap, wh, reg, K):
    scores, inds, gidx = _nms_topk(fmap)
    scores = scores.reshape(B, KPAD)
    x1, y1, x2, y2, cls = _sc_decode(
        wh.reshape(B * 2 * HW), reg.reshape(B * 2 * HW),
        inds.reshape(B * KPAD), gidx.reshape(B * KPAD))
    x1, y1, x2, y2, cls = (a.reshape(B, KPAD) for a in (x1, y1, x2, y2, cls))
    k_zero = jnp.asarray(K, jnp.float32) - jnp.float32(K_STATIC)
    bboxes = jnp.stack([x1, y1, x2, y2], axis=2)[:, :K_STATIC, :]
    scores_out = scores[:, :K_STATIC, None] + k_zero
    clses = cls[:, :K_STATIC, None]
    return bboxes, scores_out, clses
